# baseline (speedup 1.0000x reference)
import jax
import jax.numpy as jnp
from jax.experimental import pallas as pl
from jax.experimental.pallas import tpu as pltpu

_DIM = 2048
_NE = 8
_MAXIT = 3
_B = 4
_S = 4096
_SBLK = 512
_NS = _S // _SBLK


def _body(it_ref, w1_ref, b1_ref, w2_ref, idx_ref, y0_ref, yp0_ref, an0_ref,
          y1_ref, yp1_ref, an1_ref, out_ref, acc_ref):
    b = pl.program_id(0)
    s = pl.program_id(1)

    y0 = y0_ref[0]
    d0 = y0 - yp0_ref[0]
    a0 = y0 - an0_ref[0]
    y1 = y1_ref[0]
    d1 = y1 - yp1_ref[0]
    a1 = y1 - an1_ref[0]
    dn = (jnp.sqrt(jnp.sum(d0 * d0, axis=1, keepdims=True))
          + jnp.sqrt(jnp.sum(d1 * d1, axis=1, keepdims=True)))
    an = (jnp.sqrt(jnp.sum(a0 * a0, axis=1, keepdims=True))
          + jnp.sqrt(jnp.sum(a1 * a1, axis=1, keepdims=True)))

    @pl.when(s == 0)
    def _():
        acc_ref[0, b] = 0.0
        acc_ref[1, b] = 0.0

    acc_ref[0, b] += jnp.sum(dn)
    acc_ref[1, b] += jnp.sum(an)

    @pl.when((b == _B - 1) & (s == _NS // 2 - 1))
    def _():
        bi = jax.lax.broadcasted_iota(jnp.int32, (_B, 1), 0)
        delta = jnp.zeros((_B, 1), jnp.float32)
        drift = jnp.zeros((_B, 1), jnp.float32)
        for bb in range(_B):
            delta = jnp.where(bi == bb, acc_ref[0, bb], delta)
            drift = jnp.where(bi == bb, acc_ref[1, bb], drift)
        delta = delta * (1.0 / _S)
        drift = drift * (1.0 / _S)
        delta = delta / (jnp.mean(delta) + 1e-8)
        drift = drift / (jnp.mean(drift) + 1e-8)
        clamped = jnp.minimum(idx_ref[...], _MAXIT - 1)
        sel = (jax.lax.broadcasted_iota(jnp.int32, (1, _MAXIT), 1) == clamped
               ).astype(jnp.float32)
        emb = jax.lax.dot_general(sel, it_ref[...], (((1,), (0,)), ((), ())),
                                  preferred_element_type=jnp.float32)
        emb4 = jnp.broadcast_to(emb, (_B, 16))
        x = jnp.concatenate([delta, drift, emb4], axis=1)
        h = jax.lax.dot_general(x, w1_ref[...], (((1,), (1,)), ((), ())),
                                preferred_element_type=jnp.float32) + b1_ref[...]
        xh = h[:, : _NE * 4]
        gate = h[:, _NE * 4:]
        h2 = (gate * jax.lax.logistic(gate)) * xh
        logits = jax.lax.dot_general(h2, w2_ref[...], (((1,), (1,)), ((), ())),
                                     preferred_element_type=jnp.float32)
        mx = jnp.max(logits, axis=1, keepdims=True)
        iota = jax.lax.broadcasted_iota(jnp.int32, (_B, _NE), 1)
        first = jnp.min(jnp.where(logits == mx, iota, _NE), axis=1,
                        keepdims=True)
        out_ref[...] = (iota == first).astype(jnp.float32)


def kernel(y, y_prev, linguistic_anchor, iter_table, W1, b1, W2, iter_idx):
    iidx = jnp.asarray(iter_idx, jnp.int32).reshape(1, 1)
    return pl.pallas_call(
        _body,
        grid=(_B, _NS // 2),
        in_specs=[
            pl.BlockSpec((_MAXIT, 16), lambda b, s: (0, 0)),
            pl.BlockSpec((64, 18), lambda b, s: (0, 0)),
            pl.BlockSpec((1, 64), lambda b, s: (0, 0)),
            pl.BlockSpec((_NE, 32), lambda b, s: (0, 0)),
            pl.BlockSpec((1, 1), lambda b, s: (0, 0)),
            pl.BlockSpec((1, _SBLK, _DIM), lambda b, s: (b, 2 * s, 0)),
            pl.BlockSpec((1, _SBLK, _DIM), lambda b, s: (b, 2 * s, 0)),
            pl.BlockSpec((1, _SBLK, _DIM), lambda b, s: (b, 2 * s, 0)),
            pl.BlockSpec((1, _SBLK, _DIM), lambda b, s: (b, 2 * s + 1, 0)),
            pl.BlockSpec((1, _SBLK, _DIM), lambda b, s: (b, 2 * s + 1, 0)),
            pl.BlockSpec((1, _SBLK, _DIM), lambda b, s: (b, 2 * s + 1, 0)),
        ],
        out_specs=pl.BlockSpec((_B, _NE), lambda b, s: (0, 0)),
        out_shape=jax.ShapeDtypeStruct((_B, _NE), jnp.float32),
        scratch_shapes=[pltpu.SMEM((2, _B), jnp.float32)],
    )(iter_table, W1, b1.reshape(1, 64), W2, iidx, y, y_prev,
      linguistic_anchor, y, y_prev, linguistic_anchor)


# speedup vs baseline: 1.0258x; 1.0258x over previous
import jax
import jax.numpy as jnp
from jax.experimental import pallas as pl
from jax.experimental.pallas import tpu as pltpu

_DIM = 2048
_NE = 8
_MAXIT = 3
_B = 4
_S = 4096
_SBLK = 512
_NS = _S // _SBLK


def _body(it_ref, w1_ref, b1_ref, w2_ref, idx_ref, y_ref, yp_ref, an_ref,
          out_ref, acc_ref):
    b = pl.program_id(0)
    s = pl.program_id(1)

    y = y_ref[0]
    d = y - yp_ref[0]
    a = y - an_ref[0]
    dn = jnp.sqrt(jnp.sum(d * d, axis=1, keepdims=True))
    an = jnp.sqrt(jnp.sum(a * a, axis=1, keepdims=True))

    @pl.when(s == 0)
    def _():
        acc_ref[0, b] = 0.0
        acc_ref[1, b] = 0.0

    acc_ref[0, b] += jnp.sum(dn)
    acc_ref[1, b] += jnp.sum(an)

    @pl.when((b == _B - 1) & (s == _NS - 1))
    def _():
        bi = jax.lax.broadcasted_iota(jnp.int32, (_B, 1), 0)
        delta = jnp.zeros((_B, 1), jnp.float32)
        drift = jnp.zeros((_B, 1), jnp.float32)
        for bb in range(_B):
            delta = jnp.where(bi == bb, acc_ref[0, bb], delta)
            drift = jnp.where(bi == bb, acc_ref[1, bb], drift)
        delta = delta * (1.0 / _S)
        drift = drift * (1.0 / _S)
        delta = delta / (jnp.mean(delta) + 1e-8)
        drift = drift / (jnp.mean(drift) + 1e-8)
        clamped = jnp.minimum(idx_ref[...], _MAXIT - 1)
        sel = (jax.lax.broadcasted_iota(jnp.int32, (1, _MAXIT), 1) == clamped
               ).astype(jnp.float32)
        emb = jax.lax.dot_general(sel, it_ref[...], (((1,), (0,)), ((), ())),
                                  preferred_element_type=jnp.float32)
        emb4 = jnp.broadcast_to(emb, (_B, 16))
        x = jnp.concatenate([delta, drift, emb4], axis=1)
        h = jax.lax.dot_general(x, w1_ref[...], (((1,), (1,)), ((), ())),
                                preferred_element_type=jnp.float32) + b1_ref[...]
        xh = h[:, : _NE * 4]
        gate = h[:, _NE * 4:]
        h2 = (gate * jax.lax.logistic(gate)) * xh
        logits = jax.lax.dot_general(h2, w2_ref[...], (((1,), (1,)), ((), ())),
                                     preferred_element_type=jnp.float32)
        mx = jnp.max(logits, axis=1, keepdims=True)
        iota = jax.lax.broadcasted_iota(jnp.int32, (_B, _NE), 1)
        first = jnp.min(jnp.where(logits == mx, iota, _NE), axis=1,
                        keepdims=True)
        out_ref[...] = (iota == first).astype(jnp.float32)


def kernel(y, y_prev, linguistic_anchor, iter_table, W1, b1, W2, iter_idx):
    iidx = jnp.asarray(iter_idx, jnp.int32).reshape(1, 1)
    return pl.pallas_call(
        _body,
        grid=(_B, _NS),
        in_specs=[
            pl.BlockSpec((_MAXIT, 16), lambda b, s: (0, 0)),
            pl.BlockSpec((64, 18), lambda b, s: (0, 0)),
            pl.BlockSpec((1, 64), lambda b, s: (0, 0)),
            pl.BlockSpec((_NE, 32), lambda b, s: (0, 0)),
            pl.BlockSpec((1, 1), lambda b, s: (0, 0)),
            pl.BlockSpec((1, _SBLK, _DIM), lambda b, s: (b, s, 0)),
            pl.BlockSpec((1, _SBLK, _DIM), lambda b, s: (b, s, 0)),
            pl.BlockSpec((1, _SBLK, _DIM), lambda b, s: (b, s, 0)),
        ],
        out_specs=pl.BlockSpec((_B, _NE), lambda b, s: (0, 0)),
        out_shape=jax.ShapeDtypeStruct((_B, _NE), jnp.float32),
        scratch_shapes=[pltpu.SMEM((2, _B), jnp.float32)],
    )(iter_table, W1, b1.reshape(1, 64), W2, iidx, y, y_prev,
      linguistic_anchor)
